# baseline (speedup 1.0000x reference)
import jax
import jax.numpy as jnp
from jax.experimental import pallas as pl
from jax.experimental.pallas import tpu as pltpu

B, N, HD, NFD, EFD, TD, L = 16, 64, 128, 6, 15, 4, 3
NN = N * N
_BN_SCALE = float(1.0 / (1.0 + 1e-5) ** 0.5)
_F32 = jnp.float32
BB = 8


def _dot(a, b):
    return jnp.dot(a, b, preferred_element_type=_F32,
                   precision=jax.lax.Precision.HIGHEST)


def _dotx(a, b):
    return jnp.dot(a, b, preferred_element_type=_F32,
                   precision=jax.lax.Precision.HIGHEST)


def _sgcn_kernel(etT_ref, nf_ref, adj_ref,
                 embW_ref, embb_ref, nbg_ref, nbb_ref,
                 msgW_ref, msgb_ref, updW_ref, updb_ref, bng_ref, bnb_ref,
                 W1_ref, b1_ref, l1g_ref, l1b_ref,
                 W2_ref, b2_ref, l2g_ref, l2b_ref,
                 w3_ref, b3_ref,
                 out_ref):
    adj = adj_ref[...]
    mask = (adj > 0).astype(_F32)
    maskT = mask.T
    deg = jnp.sum(mask, axis=1, keepdims=True)

    O1 = jnp.full((128, 128), 1.0 / 128, _F32)
    O2 = jnp.full((64, 64), 1.0 / 64, _F32)
    ones64 = jnp.full((64, 64), 1.0, _F32)
    pr = jax.lax.broadcasted_iota(jnp.int32, (NN, N), 0)
    cl = jax.lax.broadcasted_iota(jnp.int32, (NN, N), 1)
    CT = (pr // N == cl).astype(_F32)
    EB = (pr % N == cl).astype(_F32)
    lr = jax.lax.broadcasted_iota(jnp.int32, (N, NN), 0)
    lc = jax.lax.broadcasted_iota(jnp.int32, (N, NN), 1)
    CF = (lc // N == lr).astype(_F32)

    for bb in range(BB):
        nf = nf_ref[bb]
        emb = _dot(nf, embW_ref[...].T) + embb_ref[...]
        emb = emb * (_BN_SCALE * nbg_ref[...]) + nbb_ref[...]
        emb = jnp.maximum(emb, 0.0)

        et = etT_ref[bb].T
        er3 = et[:, :3].reshape(N, N, 3)
        er_sum = jnp.sum(er3 * maskT[:, :, None], axis=0)

        for l in range(L):
            Wl = msgW_ref[l]
            A = _dot(emb, Wl[:, :HD].T)
            msgs = _dot(mask, A)
            msgs = msgs + _dot(er_sum, Wl[:, HD:HD + 3].T)
            msgs = msgs + deg * msgb_ref[l][None, :]
            Ul = updW_ref[l]
            upd = _dot(emb, Ul[:, :HD].T) + _dot(msgs, Ul[:, HD:].T) \
                + updb_ref[l][None, :]
            upd = jnp.maximum(upd, 0.0)
            upd = upd * (_BN_SCALE * bng_ref[l][None, :]) + bnb_ref[l][None, :]
            emb = upd + emb

        W1 = W1_ref[...]
        P = _dot(emb, W1[:, :HD].T) + b1_ref[...]
        Q = _dot(emb, W1[:, HD:2 * HD].T)
        R = _dot(et, W1[:, 2 * HD:].T)
        h = R + _dotx(CT, P) + _dotx(EB, Q)
        h = h - _dot(h, O1)
        v = _dot(h * h, O1)
        h = h * jax.lax.rsqrt(v + 1e-5) * l1g_ref[...] + l1b_ref[...]
        h = jnp.maximum(h, 0.0)

        h = _dot(h, W2_ref[...].T) + b2_ref[...]
        h = h - _dot(h, O2)
        v = _dot(h * h, O2)
        h = h * jax.lax.rsqrt(v + 1e-5) * l2g_ref[...] + l2b_ref[...]
        h = jnp.maximum(h, 0.0)

        rs = _dotx(h * w3_ref[...], ones64)
        Z = rs * EB
        out2d = _dotx(CF, Z) + b3_ref[0, 0]
        out_ref[bb] = jnp.maximum(out2d, 0.0)


def kernel(node_features, edge_features, temporal_features, adjacency,
           emb_W, emb_b, node_bn_g, node_bn_b,
           msg_W, msg_b, upd_W, upd_b, bn_g, bn_b,
           fp_W1, fp_b1, ln1_g, ln1_b,
           fp_W2, fp_b2, ln2_g, ln2_b,
           fp_W3, fp_b3):
    etT = jnp.concatenate(
        [edge_features[..., EFD - 3:], temporal_features], axis=-1
    ).transpose(0, 3, 1, 2).reshape(B, 3 + TD, NN)
    row = lambda x: x.reshape(1, -1)

    full = lambda shape: pl.BlockSpec(shape, lambda b: (0,) * len(shape))
    out = pl.pallas_call(
        _sgcn_kernel,
        grid=(B // BB,),
        in_specs=[
            pl.BlockSpec((BB, 3 + TD, NN), lambda b: (b, 0, 0)),
            pl.BlockSpec((BB, N, NFD), lambda b: (b, 0, 0)),
            full((N, N)),
            full((HD, NFD)), full((1, HD)), full((1, HD)), full((1, HD)),
            full((L, HD, HD + 3)), full((L, HD)),
            full((L, HD, 2 * HD)), full((L, HD)),
            full((L, HD)), full((L, HD)),
            full((128, 2 * HD + 3 + TD)), full((1, 128)),
            full((1, 128)), full((1, 128)),
            full((64, 128)), full((1, 64)), full((1, 64)), full((1, 64)),
            full((1, 64)), full((1, 1)),
        ],
        out_specs=pl.BlockSpec((BB, N, N), lambda b: (b, 0, 0)),
        out_shape=jax.ShapeDtypeStruct((B, N, N), _F32),
    )(etT, node_features, adjacency,
      emb_W, row(emb_b), row(node_bn_g), row(node_bn_b),
      msg_W, msg_b, upd_W, upd_b, bn_g, bn_b,
      fp_W1, row(fp_b1), row(ln1_g), row(ln1_b),
      fp_W2, row(fp_b2), row(ln2_g), row(ln2_b),
      fp_W3, row(fp_b3).reshape(1, 1))
    return out


# speedup vs baseline: 2.8940x; 2.8940x over previous
import jax
import jax.numpy as jnp
from jax.experimental import pallas as pl
from jax.experimental.pallas import tpu as pltpu

B, N, HD, NFD, EFD, TD, L = 16, 64, 128, 6, 15, 4, 3
NN = N * N
BB = 8
_BN_SCALE = float(1.0 / (1.0 + 1e-5) ** 0.5)
_F32 = jnp.float32
_BF16 = jnp.bfloat16


def _d(a, b):
    return jnp.dot(a, b, preferred_element_type=_F32)


def _split(x):
    hi = x.astype(_BF16).astype(_F32)
    return hi, x - hi


def _dot3(a, b):
    ah, al = _split(a)
    bh, bl = _split(b)
    return _d(ah, bh) + _d(ah, bl) + _d(al, bh)


def _dot2(a, b):
    ah, al = _split(a)
    return _d(ah, b) + _d(al, b)


def _dot2r(a, b):
    bh, bl = _split(b)
    return _d(a, bh) + _d(a, bl)


def _sgcn_kernel(etT_ref, nf_ref, adj_ref,
                 embW_ref, embb_ref, nbg_ref, nbb_ref,
                 msgW_ref, msgb_ref, updW_ref, updb_ref, bng_ref, bnb_ref,
                 W1_ref, b1_ref, l1g_ref, l1b_ref,
                 W2_ref, b2_ref, l2g_ref, l2b_ref,
                 w3_ref, b3_ref,
                 out_ref):
    adj = adj_ref[...]
    mask = (adj > 0).astype(_F32)
    maskT = mask.T
    deg = jnp.sum(mask, axis=1, keepdims=True)

    O1 = jnp.full((128, 128), 1.0 / 128, _F32)
    O2 = jnp.full((64, 64), 1.0 / 64, _F32)
    ones64 = jnp.full((64, 64), 1.0, _F32)
    pr = jax.lax.broadcasted_iota(jnp.int32, (NN, N), 0)
    cl = jax.lax.broadcasted_iota(jnp.int32, (NN, N), 1)
    CT = (pr // N == cl).astype(_F32)
    EB = (pr % N == cl).astype(_F32)
    lr = jax.lax.broadcasted_iota(jnp.int32, (N, NN), 0)
    lc = jax.lax.broadcasted_iota(jnp.int32, (N, NN), 1)
    CF = (lc // N == lr).astype(_F32)

    for bb in range(BB):
        nf = nf_ref[bb]
        emb = _dot3(nf, embW_ref[...].T) + embb_ref[...]
        emb = emb * (_BN_SCALE * nbg_ref[...]) + nbb_ref[...]
        emb = jnp.maximum(emb, 0.0)

        et = etT_ref[bb].T
        er3 = et[:, :3].reshape(N, N, 3)
        er_sum = jnp.sum(er3 * maskT[:, :, None], axis=0)

        for l in range(L):
            Wl = msgW_ref[l]
            A = _dot3(emb, Wl[:, :HD].T)
            msgs = _dot2r(mask, A)
            msgs = msgs + _dot3(er_sum, Wl[:, HD:HD + 3].T)
            msgs = msgs + deg * msgb_ref[l][None, :]
            Ul = updW_ref[l]
            upd = _dot3(emb, Ul[:, :HD].T) + _dot3(msgs, Ul[:, HD:].T) \
                + updb_ref[l][None, :]
            upd = jnp.maximum(upd, 0.0)
            upd = upd * (_BN_SCALE * bng_ref[l][None, :]) + bnb_ref[l][None, :]
            emb = upd + emb

        W1 = W1_ref[...]
        P = _dot3(emb, W1[:, :HD].T) + b1_ref[...]
        Q = _dot3(emb, W1[:, HD:2 * HD].T)
        R = _dot3(et, W1[:, 2 * HD:].T)
        h = R + _dot2r(CT, P) + _dot2r(EB, Q)
        v = _d(h * h, O1)
        h = h * jax.lax.rsqrt(v + 1e-5) * l1g_ref[...] + l1b_ref[...]
        h = jnp.maximum(h, 0.0)

        h = _dot3(h, W2_ref[...].T) + b2_ref[...]
        v = _d(h * h, O2)
        h = h * jax.lax.rsqrt(v + 1e-5) * l2g_ref[...] + l2b_ref[...]
        h = jnp.maximum(h, 0.0)

        rs = _dot2(h * w3_ref[...], ones64)
        Z = rs * EB
        out2d = _dot2r(CF, Z) + b3_ref[0, 0]
        out_ref[bb] = jnp.maximum(out2d, 0.0)


def kernel(node_features, edge_features, temporal_features, adjacency,
           emb_W, emb_b, node_bn_g, node_bn_b,
           msg_W, msg_b, upd_W, upd_b, bn_g, bn_b,
           fp_W1, fp_b1, ln1_g, ln1_b,
           fp_W2, fp_b2, ln2_g, ln2_b,
           fp_W3, fp_b3):
    etT = jnp.concatenate(
        [edge_features[..., EFD - 3:], temporal_features], axis=-1
    ).transpose(0, 3, 1, 2).reshape(B, 3 + TD, NN)
    W1c = fp_W1 - jnp.mean(fp_W1, axis=0, keepdims=True)
    b1c = fp_b1 - jnp.mean(fp_b1)
    W2c = fp_W2 - jnp.mean(fp_W2, axis=0, keepdims=True)
    b2c = fp_b2 - jnp.mean(fp_b2)
    row = lambda x: x.reshape(1, -1)

    full = lambda shape: pl.BlockSpec(shape, lambda b: (0,) * len(shape))
    out = pl.pallas_call(
        _sgcn_kernel,
        grid=(B // BB,),
        in_specs=[
            pl.BlockSpec((BB, 3 + TD, NN), lambda b: (b, 0, 0)),
            pl.BlockSpec((BB, N, NFD), lambda b: (b, 0, 0)),
            full((N, N)),
            full((HD, NFD)), full((1, HD)), full((1, HD)), full((1, HD)),
            full((L, HD, HD + 3)), full((L, HD)),
            full((L, HD, 2 * HD)), full((L, HD)),
            full((L, HD)), full((L, HD)),
            full((128, 2 * HD + 3 + TD)), full((1, 128)),
            full((1, 128)), full((1, 128)),
            full((64, 128)), full((1, 64)), full((1, 64)), full((1, 64)),
            full((1, 64)), full((1, 1)),
        ],
        out_specs=pl.BlockSpec((BB, N, N), lambda b: (b, 0, 0)),
        out_shape=jax.ShapeDtypeStruct((B, N, N), _F32),
    )(etT, node_features, adjacency,
      emb_W, row(emb_b), row(node_bn_g), row(node_bn_b),
      msg_W, msg_b, upd_W, upd_b, bn_g, bn_b,
      W1c, row(b1c), row(ln1_g), row(ln1_b),
      W2c, row(b2c), row(ln2_g), row(ln2_b),
      fp_W3, row(fp_b3).reshape(1, 1))
    return out


# speedup vs baseline: 3.3338x; 1.1520x over previous
import jax
import jax.numpy as jnp
from jax.experimental import pallas as pl
from jax.experimental.pallas import tpu as pltpu

B, N, HD, NFD, EFD, TD, L = 16, 64, 128, 6, 15, 4, 3
NN = N * N
BB = 8
_BN_SCALE = float(1.0 / (1.0 + 1e-5) ** 0.5)
_F32 = jnp.float32
_BF16 = jnp.bfloat16


def _d(a, b):
    return jnp.dot(a, b, preferred_element_type=_F32)


def _split(x):
    hi = x.astype(_BF16).astype(_F32)
    return hi, x - hi


def _dot3p(x, w_hi, w_lo):
    xh, xl = _split(x)
    return _d(xh, w_hi) + _d(xh, w_lo) + _d(xl, w_hi)


def _dot3s(xh, xl, w_hi, w_lo):
    return _d(xh, w_hi) + _d(xh, w_lo) + _d(xl, w_hi)


def _dot2(a, b):
    ah, al = _split(a)
    return _d(ah, b) + _d(al, b)


def _dot2r(a, b):
    bh, bl = _split(b)
    return _d(a, bh) + _d(a, bl)


def _sgcn_kernel(eth_ref, etl_ref, nf_ref, adj_ref,
                 embWh_ref, embWl_ref, embb_ref, nbg_ref, nbb_ref,
                 WhTh_ref, WhTl_ref, WeTh_ref, WeTl_ref, msgb_ref,
                 WueTh_ref, WueTl_ref, WumTh_ref, WumTl_ref, updb_ref,
                 bng_ref, bnb_ref,
                 W1aTh_ref, W1aTl_ref, W1bTh_ref, W1bTl_ref,
                 W1cTh_ref, W1cTl_ref, b1_ref, l1g_ref, l1b_ref,
                 W2Th_ref, W2Tl_ref, b2_ref, l2g_ref, l2b_ref,
                 w3_ref, b3_ref,
                 out_ref):
    adj = adj_ref[...]
    mask = (adj > 0).astype(_F32)
    maskT = mask.T
    deg = jnp.sum(mask, axis=1, keepdims=True)

    O1 = jnp.full((128, 128), 1.0 / 128, _F32)
    O2 = jnp.full((64, 64), 1.0 / 64, _F32)
    ones64 = jnp.full((64, 64), 1.0, _F32)
    pr = jax.lax.broadcasted_iota(jnp.int32, (NN, N), 0)
    cl = jax.lax.broadcasted_iota(jnp.int32, (NN, N), 1)
    EB = (pr % N == cl).astype(_F32)
    lr = jax.lax.broadcasted_iota(jnp.int32, (N, NN), 0)
    lc = jax.lax.broadcasted_iota(jnp.int32, (N, NN), 1)
    CF = (lc // N == lr).astype(_F32)

    for bb in range(BB):
        nf = nf_ref[bb]
        emb = _dot3p(nf, embWh_ref[...], embWl_ref[...]) + embb_ref[...]
        emb = emb * (_BN_SCALE * nbg_ref[...]) + nbb_ref[...]
        emb = jnp.maximum(emb, 0.0)

        eth = eth_ref[bb].T
        etl = etl_ref[bb].T
        er3 = (eth[:, :3] + etl[:, :3]).reshape(N, N, 3)
        er_sum = jnp.sum(er3 * maskT[:, :, None], axis=0)

        for l in range(L):
            A = _dot3p(emb, WhTh_ref[l], WhTl_ref[l])
            msgs = _dot2r(mask, A)
            msgs = msgs + _dot3p(er_sum, WeTh_ref[l], WeTl_ref[l])
            msgs = msgs + deg * msgb_ref[l][None, :]
            upd = _dot3p(emb, WueTh_ref[l], WueTl_ref[l]) \
                + _dot3p(msgs, WumTh_ref[l], WumTl_ref[l]) \
                + updb_ref[l][None, :]
            upd = jnp.maximum(upd, 0.0)
            upd = upd * (_BN_SCALE * bng_ref[l][None, :]) + bnb_ref[l][None, :]
            emb = upd + emb

        P = _dot3p(emb, W1aTh_ref[...], W1aTl_ref[...]) + b1_ref[...]
        Q = _dot3p(emb, W1bTh_ref[...], W1bTl_ref[...])
        R = _dot3s(eth, etl, W1cTh_ref[...], W1cTl_ref[...])
        h = R.reshape(N, N, 128) + P[:, None, :] + Q[None, :, :]
        h = h.reshape(NN, 128)
        v = _d(h * h, O1)
        h = h * jax.lax.rsqrt(v + 1e-5) * l1g_ref[...] + l1b_ref[...]
        h = jnp.maximum(h, 0.0)

        h = _dot3p(h, W2Th_ref[...], W2Tl_ref[...]) + b2_ref[...]
        v = _d(h * h, O2)
        h = h * jax.lax.rsqrt(v + 1e-5) * l2g_ref[...] + l2b_ref[...]
        h = jnp.maximum(h, 0.0)

        rs = _dot2(h * w3_ref[...], ones64)
        Z = rs * EB
        out2d = _dot2r(CF, Z) + b3_ref[0, 0]
        out_ref[bb] = jnp.maximum(out2d, 0.0)


def kernel(node_features, edge_features, temporal_features, adjacency,
           emb_W, emb_b, node_bn_g, node_bn_b,
           msg_W, msg_b, upd_W, upd_b, bn_g, bn_b,
           fp_W1, fp_b1, ln1_g, ln1_b,
           fp_W2, fp_b2, ln2_g, ln2_b,
           fp_W3, fp_b3):
    etT = jnp.concatenate(
        [edge_features[..., EFD - 3:], temporal_features], axis=-1
    ).transpose(0, 3, 1, 2).reshape(B, 3 + TD, NN)
    etTh = etT.astype(_BF16).astype(_F32)
    etTl = etT - etTh
    W1c = fp_W1 - jnp.mean(fp_W1, axis=0, keepdims=True)
    b1c = fp_b1 - jnp.mean(fp_b1)
    W2c = fp_W2 - jnp.mean(fp_W2, axis=0, keepdims=True)
    b2c = fp_b2 - jnp.mean(fp_b2)

    def tsplit(w):
        wT = jnp.swapaxes(w, -1, -2)
        hi = wT.astype(_BF16).astype(_F32)
        return hi, wT - hi

    embWh, embWl = tsplit(emb_W)
    WhTh, WhTl = tsplit(msg_W[:, :, :HD])
    WeTh, WeTl = tsplit(msg_W[:, :, HD:])
    WueTh, WueTl = tsplit(upd_W[:, :, :HD])
    WumTh, WumTl = tsplit(upd_W[:, :, HD:])
    W1aTh, W1aTl = tsplit(W1c[:, :HD])
    W1bTh, W1bTl = tsplit(W1c[:, HD:2 * HD])
    W1cTh, W1cTl = tsplit(W1c[:, 2 * HD:])
    W2Th, W2Tl = tsplit(W2c)
    row = lambda x: x.reshape(1, -1)

    full = lambda shape: pl.BlockSpec(shape, lambda b: (0,) * len(shape))
    out = pl.pallas_call(
        _sgcn_kernel,
        grid=(B // BB,),
        in_specs=[
            pl.BlockSpec((BB, 3 + TD, NN), lambda b: (b, 0, 0)),
            pl.BlockSpec((BB, 3 + TD, NN), lambda b: (b, 0, 0)),
            pl.BlockSpec((BB, N, NFD), lambda b: (b, 0, 0)),
            full((N, N)),
            full((NFD, HD)), full((NFD, HD)), full((1, HD)), full((1, HD)),
            full((1, HD)),
            full((L, HD, HD)), full((L, HD, HD)),
            full((L, 3, HD)), full((L, 3, HD)), full((L, HD)),
            full((L, HD, HD)), full((L, HD, HD)),
            full((L, HD, HD)), full((L, HD, HD)), full((L, HD)),
            full((L, HD)), full((L, HD)),
            full((HD, 128)), full((HD, 128)), full((HD, 128)), full((HD, 128)),
            full((3 + TD, 128)), full((3 + TD, 128)),
            full((1, 128)), full((1, 128)), full((1, 128)),
            full((128, 64)), full((128, 64)),
            full((1, 64)), full((1, 64)), full((1, 64)),
            full((1, 64)), full((1, 1)),
        ],
        out_specs=pl.BlockSpec((BB, N, N), lambda b: (b, 0, 0)),
        out_shape=jax.ShapeDtypeStruct((B, N, N), _F32),
    )(etTh, etTl, node_features, adjacency,
      embWh, embWl, row(emb_b), row(node_bn_g), row(node_bn_b),
      WhTh, WhTl, WeTh, WeTl, msg_b,
      WueTh, WueTl, WumTh, WumTl, upd_b,
      bn_g, bn_b,
      W1aTh, W1aTl, W1bTh, W1bTl, W1cTh, W1cTl,
      row(b1c), row(ln1_g), row(ln1_b),
      W2Th, W2Tl, row(b2c), row(ln2_g), row(ln2_b),
      fp_W3, row(fp_b3).reshape(1, 1))
    return out
